# baseline (speedup 1.0000x reference)
import functools

import jax
import jax.numpy as jnp
import numpy as np
from jax import lax
from jax.experimental import pallas as pl
from jax.experimental.pallas import tpu as pltpu
from jax.experimental.pallas import tpu_sc as plsc

_N = 10000
_E = 320000
_IN = 128
_HID = 128
_OUT = 64

_NC = 2
_NS = 16
_L = 16
_NW = _NC * _NS


_BM = 10000


def _mm1_body(x_ref, w_ref, b_ref, o_ref):
    o_ref[...] = jnp.dot(x_ref[...], w_ref[...],
                         preferred_element_type=jnp.float32) + b_ref[...]


def _linear1(x, W1, b1):
    return pl.pallas_call(
        _mm1_body,
        grid=(_N // _BM,),
        in_specs=[pl.BlockSpec((_BM, _IN), lambda i: (i, 0)),
                  pl.BlockSpec((_IN, _HID), lambda i: (0, 0)),
                  pl.BlockSpec((1, _HID), lambda i: (0, 0))],
        out_specs=pl.BlockSpec((_BM, _HID), lambda i: (i, 0)),
        out_shape=jax.ShapeDtypeStruct((_N, _HID), jnp.float32),
    )(x, W1, b1[None])


def _mm2_body(p0_ref, p1_ref, w_ref, b_ref, o_ref):
    h = jnp.maximum(p0_ref[...] + p1_ref[...], 0.0)
    res = jnp.dot(h, w_ref[...],
                  preferred_element_type=jnp.float32) + b_ref[...]
    o_ref[0] = res[:, :_OUT // 2]
    o_ref[1] = res[:, _OUT // 2:]


def _linear2(p0, p1, W2, b2):
    return pl.pallas_call(
        _mm2_body,
        grid=(_N // _BM,),
        in_specs=[pl.BlockSpec((_BM, _HID), lambda i: (i, 0)),
                  pl.BlockSpec((_BM, _HID), lambda i: (i, 0)),
                  pl.BlockSpec((_HID, _OUT), lambda i: (0, 0)),
                  pl.BlockSpec((1, _OUT), lambda i: (0, 0))],
        out_specs=pl.BlockSpec((2, _BM, _OUT // 2), lambda i: (0, i, 0)),
        out_shape=jax.ShapeDtypeStruct((2, _N, _OUT // 2), jnp.float32),
    )(p0, p1, W2, b2[None])


def _add_body(a_ref, b_ref, o_ref):
    o_ref[...] = a_ref[...] + b_ref[...]


def _final_add(a, b):
    m, n = a.shape
    bm = m
    return pl.pallas_call(
        _add_body,
        grid=(1,),
        in_specs=[pl.BlockSpec((bm, n), lambda i: (i, 0)),
                  pl.BlockSpec((bm, n), lambda i: (i, 0))],
        out_specs=pl.BlockSpec((bm, n), lambda i: (i, 0)),
        out_shape=jax.ShapeDtypeStruct((m, n), jnp.float32),
    )(a, b)


def _make_spmm(D):
    EW = _E // _NW
    CH = 80
    NCH = EW // CH
    G = 25
    NSUP = NCH // G
    NF = D // _L
    RPT = 632
    RPT_LAST = _N - (_NS - 1) * RPT

    mesh = plsc.VectorSubcoreMesh(core_axis_name="c", subcore_axis_name="s",
                                  num_cores=_NC)

    @functools.partial(
        pl.kernel, mesh=mesh,
        compiler_params=pltpu.CompilerParams(use_tc_tiling_on_sc=False),
        out_type=jax.ShapeDtypeStruct((_NC, _N, D), jnp.float32),
        scratch_types=[
            pltpu.VMEM_SHARED((_N, D), jnp.float32),
            pltpu.VMEM((G, CH), jnp.int32),
            pltpu.VMEM((G, CH), jnp.int32),
            pltpu.VMEM((G, CH), jnp.float32),
            pltpu.VMEM((CH, D), jnp.float32),
            pltpu.VMEM((CH, D), jnp.float32),
            pltpu.VMEM((CH, D), jnp.float32),
            pltpu.VMEM((CH, D), jnp.float32),
        ] + [pltpu.SemaphoreType.DMA] * 4)
    def spmm(h_hbm, src_hbm, dst_hbm, w_hbm, out_hbm,
             acc, src_i, dst_i, w_i, gb0, gb1, sb0, sb1,
             g0, g1, s0, s1):
        gb = (gb0, gb1)
        sb = (sb0, sb1)
        gsem = (g0, g1)
        ssem = (s0, s1)
        c = lax.axis_index("c")
        s = lax.axis_index("s")
        wid = s * _NC + c
        rbase = s * RPT

        zero = jnp.zeros((_L,), jnp.float32)

        def zrow(i, carry):
            for j in range(D // _L):
                sb0[i, pl.ds(j * _L, _L)] = zero
            return carry
        lax.fori_loop(0, CH, zrow, 0)

        def zero_acc(nrows):
            nfull, tail = divmod(nrows, CH)

            def f():
                for k in range(nfull):
                    pltpu.sync_copy(sb0, acc.at[pl.ds(rbase + k * CH, CH)])
                if tail:
                    pltpu.sync_copy(sb0.at[pl.ds(0, tail)],
                                    acc.at[pl.ds(rbase + nfull * CH, tail)])
            return f
        pl.when(s < _NS - 1)(zero_acc(RPT))
        pl.when(s == _NS - 1)(zero_acc(RPT_LAST))
        plsc.subcore_barrier()

        def fire_gather(m, b):
            pltpu.async_copy(h_hbm.at[src_i.at[m]], gb[b], gsem[b])

        def wait_gather(m, b):
            pltpu.make_async_copy(h_hbm.at[src_i.at[m]], gb[b],
                                  gsem[b]).wait()

        def fire_scatter(m, b):
            pltpu.async_copy(sb[b], acc.at[dst_i.at[m]], ssem[b], add=True)

        def wait_scatter(m, b):
            pltpu.make_async_copy(sb[b], acc.at[dst_i.at[m]],
                                  ssem[b]).wait()

        def scale(m, b):
            gbuf = gb[b]
            sbuf = sb[b]

            @plsc.parallel_loop(0, CH // _L)
            def grp(g):
                wv16 = w_i[m, pl.ds(g * _L, _L)]
                for l in range(_L):
                    e = g * _L + l
                    wv = jnp.full((_L,), wv16[l], jnp.float32)
                    for j in range(NF):
                        sl = pl.ds(j * _L, _L)
                        sbuf[e, sl] = gbuf[e, sl] * wv

        def slot(m, b, wait_prev_scatter, gather_ahead):
            wait_gather(m, b)
            if wait_prev_scatter:
                wait_scatter(m - 2, b)
            scale(m, b)
            fire_scatter(m, b)
            if gather_ahead:
                fire_gather(m + 2, b)

        def super_body(u, carry):
            pltpu.sync_copy(src_hbm.at[wid, pl.ds(u * G, G)], src_i)
            pltpu.sync_copy(dst_hbm.at[wid, pl.ds(u * G, G)], dst_i)
            pltpu.sync_copy(w_hbm.at[wid, pl.ds(u * G, G)], w_i)

            fire_gather(0, 0)
            fire_gather(1, 1)
            slot(0, 0, False, True)
            slot(1, 1, False, True)
            slot(2, 0, True, True)

            def round_body(g, c2):
                m0 = 3 + g * 2
                slot(m0, 1, True, True)
                slot(m0 + 1, 0, True, True)
                return c2
            lax.fori_loop(0, (G - 5) // 2, round_body, 0)

            slot(G - 2, (G - 2) % 2, True, False)
            slot(G - 1, (G - 1) % 2, True, False)
            wait_scatter(G - 2, (G - 2) % 2)
            wait_scatter(G - 1, (G - 1) % 2)
            return carry
        lax.fori_loop(0, NSUP, super_body, 0)

        plsc.subcore_barrier()

        def writeback(nrows):
            def f():
                pltpu.sync_copy(acc.at[pl.ds(rbase, nrows)],
                                out_hbm.at[c, pl.ds(rbase, nrows)])
            return f
        pl.when(s < _NS - 1)(writeback(RPT))
        pl.when(s == _NS - 1)(writeback(RPT_LAST))

    return spmm


def _make_spmm_colsplit():
    DW2 = _OUT // 2
    EW = _E // _NS
    CH = 80
    NCH = EW // CH
    G = 25
    NSUP = NCH // G
    NF = DW2 // _L
    RPT = 632
    RPT_LAST = _N - (_NS - 1) * RPT

    mesh = plsc.VectorSubcoreMesh(core_axis_name="c", subcore_axis_name="s",
                                  num_cores=_NC)

    @functools.partial(
        pl.kernel, mesh=mesh,
        compiler_params=pltpu.CompilerParams(use_tc_tiling_on_sc=False),
        out_type=jax.ShapeDtypeStruct((_N, _OUT), jnp.float32),
        scratch_types=[
            pltpu.VMEM_SHARED((_N, DW2), jnp.float32),
            pltpu.VMEM((G, CH), jnp.int32),
            pltpu.VMEM((G, CH), jnp.int32),
            pltpu.VMEM((G, CH), jnp.float32),
        ] + [pltpu.VMEM((CH, DW2), jnp.float32)] * 3
          + [pltpu.VMEM((CH, DW2), jnp.float32)] * 3
          + [pltpu.SemaphoreType.DMA] * 6)
    def spmm(h_hbm, src_hbm, dst_hbm, w_hbm, out_hbm,
             acc, src_i, dst_i, w_i, gb0, gb1, gb2, sb0, sb1, sb2,
             g0, g1, g2, s0, s1, s2):
        gb = (gb0, gb1, gb2)
        sb = (sb0, sb1, sb2)
        gsem = (g0, g1, g2)
        ssem = (s0, s1, s2)
        c = lax.axis_index("c")
        s = lax.axis_index("s")
        rbase = s * RPT
        hc = h_hbm.at[c]

        zero = jnp.zeros((_L,), jnp.float32)

        def zrow(i, carry):
            for j in range(NF):
                sb0[i, pl.ds(j * _L, _L)] = zero
            return carry
        lax.fori_loop(0, CH, zrow, 0)

        def zero_acc(nrows):
            nfull, tail = divmod(nrows, CH)

            def f():
                for k in range(nfull):
                    pltpu.sync_copy(sb0, acc.at[pl.ds(rbase + k * CH, CH)])
                if tail:
                    pltpu.sync_copy(sb0.at[pl.ds(0, tail)],
                                    acc.at[pl.ds(rbase + nfull * CH, tail)])
            return f
        pl.when(s < _NS - 1)(zero_acc(RPT))
        pl.when(s == _NS - 1)(zero_acc(RPT_LAST))
        plsc.subcore_barrier()

        def fire_gather(m, b):
            pltpu.async_copy(hc.at[src_i.at[m]], gb[b], gsem[b])

        def wait_gather(m, b):
            pltpu.make_async_copy(hc.at[src_i.at[m]], gb[b], gsem[b]).wait()

        def fire_scatter(m, b):
            pltpu.async_copy(sb[b], acc.at[dst_i.at[m]], ssem[b], add=True)

        def wait_scatter(m, b):
            pltpu.make_async_copy(sb[b], acc.at[dst_i.at[m]],
                                  ssem[b]).wait()

        def scale(m, b):
            gbuf = gb[b]
            sbuf = sb[b]

            @plsc.parallel_loop(0, CH // _L)
            def grp(g):
                wv16 = w_i[m, pl.ds(g * _L, _L)]
                for l in range(_L):
                    e = g * _L + l
                    wv = jnp.full((_L,), wv16[l], jnp.float32)
                    for j in range(NF):
                        sl = pl.ds(j * _L, _L)
                        sbuf[e, sl] = gbuf[e, sl] * wv

        def slot(m, b, wait_prev_scatter, gather_ahead):
            wait_gather(m, b)
            if wait_prev_scatter:
                wait_scatter(m - 3, b)
            scale(m, b)
            fire_scatter(m, b)
            if gather_ahead:
                fire_gather(m + 2, (b + 2) % 3)

        def super_body(u, carry):
            pltpu.sync_copy(src_hbm.at[s, pl.ds(u * G, G)], src_i)
            pltpu.sync_copy(dst_hbm.at[s, pl.ds(u * G, G)], dst_i)
            pltpu.sync_copy(w_hbm.at[s, pl.ds(u * G, G)], w_i)

            fire_gather(0, 0)
            fire_gather(1, 1)
            slot(0, 0, False, True)
            slot(1, 1, False, True)
            slot(2, 2, False, True)

            def round_body(g, c2):
                m0 = 3 + g * 3
                for k in range(3):
                    slot(m0 + k, k, True, True)
                return c2
            lax.fori_loop(0, (G - 7) // 3, round_body, 0)

            slot(G - 4, (G - 4) % 3, True, True)
            slot(G - 3, (G - 3) % 3, True, True)
            slot(G - 2, (G - 2) % 3, True, False)
            slot(G - 1, (G - 1) % 3, True, False)
            for m in range(G - 3, G):
                wait_scatter(m, m % 3)
            return carry
        lax.fori_loop(0, NSUP, super_body, 0)

        plsc.subcore_barrier()

        def writeback(nrows):
            def f():
                pltpu.sync_copy(
                    acc.at[pl.ds(rbase, nrows)],
                    out_hbm.at[pl.ds(rbase, nrows), pl.ds(c * DW2, DW2)])
            return f
        pl.when(s < _NS - 1)(writeback(RPT))
        pl.when(s == _NS - 1)(writeback(RPT_LAST))

    return spmm


_spmm_cache = {}


def _spmm(D):
    if D not in _spmm_cache:
        _spmm_cache[D] = _make_spmm(D)
    return _spmm_cache[D]


def _spmm_out():
    if "out" not in _spmm_cache:
        _spmm_cache["out"] = _make_spmm_colsplit()
    return _spmm_cache["out"]


def kernel(x, edge_index, edge_weight, W1, b1, W2, b2):
    ch = 80
    srcf = edge_index[0].astype(jnp.int32)
    dstf = edge_index[1].astype(jnp.int32)
    wf = edge_weight.astype(jnp.float32)
    ew32 = _E // _NW
    src = srcf.reshape(_NW, ew32 // ch, ch)
    dst = dstf.reshape(_NW, ew32 // ch, ch)
    w = wf.reshape(_NW, ew32 // ch, ch)
    ew16 = _E // _NS
    src16 = srcf.reshape(_NS, ew16 // ch, ch)
    dst16 = dstf.reshape(_NS, ew16 // ch, ch)
    w16 = wf.reshape(_NS, ew16 // ch, ch)

    h1 = _linear1(x, W1, b1)
    p = _spmm(_HID)(h1, src, dst, w)
    h2 = _linear2(p[0], p[1], W2, b2)
    return _spmm_out()(h2, src16, dst16, w16)


# speedup vs baseline: 1.1162x; 1.1162x over previous
import functools

import jax
import jax.numpy as jnp
import numpy as np
from jax import lax
from jax.experimental import pallas as pl
from jax.experimental.pallas import tpu as pltpu
from jax.experimental.pallas import tpu_sc as plsc

_N = 10000
_E = 320000
_IN = 128
_HID = 128
_OUT = 64

_NC = 2
_NS = 16
_L = 16
_NW = _NC * _NS


_BM = 10000


def _mm1_body(x_ref, w_ref, b_ref, o_ref):
    o_ref[...] = jnp.dot(x_ref[...], w_ref[...],
                         preferred_element_type=jnp.float32) + b_ref[...]


def _linear1(x, W1, b1):
    return pl.pallas_call(
        _mm1_body,
        grid=(_N // _BM,),
        in_specs=[pl.BlockSpec((_BM, _IN), lambda i: (i, 0)),
                  pl.BlockSpec((_IN, _HID), lambda i: (0, 0)),
                  pl.BlockSpec((1, _HID), lambda i: (0, 0))],
        out_specs=pl.BlockSpec((_BM, _HID), lambda i: (i, 0)),
        out_shape=jax.ShapeDtypeStruct((_N, _HID), jnp.float32),
    )(x, W1, b1[None])


def _mm2_body(p0_ref, p1_ref, w_ref, b_ref, o_ref):
    h = jnp.maximum(p0_ref[...] + p1_ref[...], 0.0)
    o_ref[...] = jnp.dot(h, w_ref[...],
                         preferred_element_type=jnp.float32) + b_ref[...]


def _linear2(p0, p1, W2, b2):
    return pl.pallas_call(
        _mm2_body,
        grid=(_N // _BM,),
        in_specs=[pl.BlockSpec((_BM, _HID), lambda i: (i, 0)),
                  pl.BlockSpec((_BM, _HID), lambda i: (i, 0)),
                  pl.BlockSpec((_HID, _OUT), lambda i: (0, 0)),
                  pl.BlockSpec((1, _OUT), lambda i: (0, 0))],
        out_specs=pl.BlockSpec((_BM, _OUT), lambda i: (i, 0)),
        out_shape=jax.ShapeDtypeStruct((_N, _OUT), jnp.float32),
    )(p0, p1, W2, b2[None])


def _add_body(a_ref, b_ref, o_ref):
    o_ref[...] = a_ref[...] + b_ref[...]


def _final_add(a, b):
    m, n = a.shape
    bm = m
    return pl.pallas_call(
        _add_body,
        grid=(1,),
        in_specs=[pl.BlockSpec((bm, n), lambda i: (i, 0)),
                  pl.BlockSpec((bm, n), lambda i: (i, 0))],
        out_specs=pl.BlockSpec((bm, n), lambda i: (i, 0)),
        out_shape=jax.ShapeDtypeStruct((m, n), jnp.float32),
    )(a, b)


def _make_spmm(D):
    EW = _E // _NW
    CH = 80
    NCH = EW // CH
    G = 25 if D > 64 else NCH
    NSUP = NCH // G
    NF = D // _L
    RPT = 632
    RPT_LAST = _N - (_NS - 1) * RPT

    mesh = plsc.VectorSubcoreMesh(core_axis_name="c", subcore_axis_name="s",
                                  num_cores=_NC)

    @functools.partial(
        pl.kernel, mesh=mesh,
        compiler_params=pltpu.CompilerParams(use_tc_tiling_on_sc=False),
        out_type=jax.ShapeDtypeStruct((_NC, _N, D), jnp.float32),
        scratch_types=[
            pltpu.VMEM_SHARED((_N, D), jnp.float32),
            pltpu.VMEM((G, CH), jnp.int32),
            pltpu.VMEM((G, CH), jnp.int32),
            pltpu.VMEM((G, CH), jnp.float32),
            pltpu.VMEM((CH, D), jnp.float32),
            pltpu.VMEM((CH, D), jnp.float32),
            pltpu.VMEM((CH, D), jnp.float32),
            pltpu.VMEM((CH, D), jnp.float32),
        ] + [pltpu.SemaphoreType.DMA] * 4)
    def spmm(h_hbm, src_hbm, dst_hbm, w_hbm, out_hbm,
             acc, src_i, dst_i, w_i, gb0, gb1, sb0, sb1,
             g0, g1, s0, s1):
        gb = (gb0, gb1)
        sb = (sb0, sb1)
        gsem = (g0, g1)
        ssem = (s0, s1)
        c = lax.axis_index("c")
        s = lax.axis_index("s")
        wid = s * _NC + c
        rbase = s * RPT

        zero = jnp.zeros((_L,), jnp.float32)

        def zrow(i, carry):
            for j in range(D // _L):
                sb0[i, pl.ds(j * _L, _L)] = zero
            return carry
        lax.fori_loop(0, CH, zrow, 0)

        def zero_acc(nrows):
            nfull, tail = divmod(nrows, CH)

            def f():
                for k in range(nfull):
                    pltpu.sync_copy(sb0, acc.at[pl.ds(rbase + k * CH, CH)])
                if tail:
                    pltpu.sync_copy(sb0.at[pl.ds(0, tail)],
                                    acc.at[pl.ds(rbase + nfull * CH, tail)])
            return f
        pl.when(s < _NS - 1)(zero_acc(RPT))
        pl.when(s == _NS - 1)(zero_acc(RPT_LAST))
        plsc.subcore_barrier()

        def fire_gather(m, b):
            pltpu.async_copy(h_hbm.at[src_i.at[m]], gb[b], gsem[b])

        def wait_gather(m, b):
            pltpu.make_async_copy(h_hbm.at[src_i.at[m]], gb[b],
                                  gsem[b]).wait()

        def fire_scatter(m, b):
            pltpu.async_copy(sb[b], acc.at[dst_i.at[m]], ssem[b], add=True)

        def wait_scatter(m, b):
            pltpu.make_async_copy(sb[b], acc.at[dst_i.at[m]],
                                  ssem[b]).wait()

        def scale(m, b):
            gbuf = gb[b]
            sbuf = sb[b]

            @plsc.parallel_loop(0, CH // _L)
            def grp(g):
                wv16 = w_i[m, pl.ds(g * _L, _L)]
                for l in range(_L):
                    e = g * _L + l
                    wv = jnp.full((_L,), wv16[l], jnp.float32)
                    for j in range(NF):
                        sl = pl.ds(j * _L, _L)
                        sbuf[e, sl] = gbuf[e, sl] * wv

        def slot(m, b, wait_prev_scatter, gather_ahead):
            wait_gather(m, b)
            if wait_prev_scatter:
                wait_scatter(m - 2, b)
            scale(m, b)
            fire_scatter(m, b)
            if gather_ahead:
                fire_gather(m + 2, b)

        def super_body(u, carry):
            pltpu.sync_copy(src_hbm.at[wid, pl.ds(u * G, G)], src_i)
            pltpu.sync_copy(dst_hbm.at[wid, pl.ds(u * G, G)], dst_i)
            pltpu.sync_copy(w_hbm.at[wid, pl.ds(u * G, G)], w_i)

            fire_gather(0, 0)
            fire_gather(1, 1)
            slot(0, 0, False, True)
            slot(1, 1, False, True)
            slot(2, 0, True, True)

            def round_body(g, c2):
                m0 = 3 + g * 2
                slot(m0, 1, True, True)
                slot(m0 + 1, 0, True, True)
                return c2
            lax.fori_loop(0, (G - 5) // 2, round_body, 0)

            slot(G - 2, (G - 2) % 2, True, False)
            slot(G - 1, (G - 1) % 2, True, False)
            wait_scatter(G - 2, (G - 2) % 2)
            wait_scatter(G - 1, (G - 1) % 2)
            return carry
        lax.fori_loop(0, NSUP, super_body, 0)

        plsc.subcore_barrier()

        def writeback(nrows):
            def f():
                pltpu.sync_copy(acc.at[pl.ds(rbase, nrows)],
                                out_hbm.at[c, pl.ds(rbase, nrows)])
            return f
        pl.when(s < _NS - 1)(writeback(RPT))
        pl.when(s == _NS - 1)(writeback(RPT_LAST))

    return spmm


_spmm_cache = {}


def _spmm(D):
    if D not in _spmm_cache:
        _spmm_cache[D] = _make_spmm(D)
    return _spmm_cache[D]


def kernel(x, edge_index, edge_weight, W1, b1, W2, b2):
    ew_ = _E // _NW
    ch = 80
    src = edge_index[0].astype(jnp.int32).reshape(_NW, ew_ // ch, ch)
    dst = edge_index[1].astype(jnp.int32).reshape(_NW, ew_ // ch, ch)
    w = edge_weight.astype(jnp.float32).reshape(_NW, ew_ // ch, ch)

    h1 = _linear1(x, W1, b1)
    p = _spmm(_HID)(h1, src, dst, w)
    h2 = _linear2(p[0], p[1], W2, b2)
    q = _spmm(_OUT)(h2, src, dst, w)
    out = _final_add(q[0].reshape(_N // 2, 2 * _OUT),
                     q[1].reshape(_N // 2, 2 * _OUT))
    return out.reshape(_N, _OUT)
